# speedup vs baseline: 21.8434x; 21.8434x over previous
import functools

import jax
import jax.numpy as jnp
from jax import lax
from jax.experimental import pallas as pl
from jax.experimental.pallas import tpu as pltpu
from jax.experimental.pallas import tpu_sc as plsc

N = 10000
E = 320000
D = 128
NC = 2
NS = 16
NW = NC * NS
NP = 10240
RPT = NP // NS

CHUNK = 128
KB = 16
EP = 327680
CPT = EP // (NW * CHUNK)
SW = CPT // KB

_MESH = plsc.VectorSubcoreMesh(
    core_axis_name="c", subcore_axis_name="s", num_cores=NC, num_subcores=NS
)


@functools.partial(
    pl.kernel,
    out_type=jax.ShapeDtypeStruct((NC, NP), jnp.float32),
    mesh=_MESH,
    scratch_types=[
        pltpu.VMEM((KB, CHUNK), jnp.int32),
        pltpu.VMEM((CHUNK,), jnp.float32),
        pltpu.VMEM((RPT,), jnp.float32),
        pltpu.VMEM_SHARED((NP,), jnp.float32)
    ],
)
def _sc_deg(dst_hbm, out_hbm, dst_v, ones_v, zbuf, acc):
    c = lax.axis_index("c")
    s = lax.axis_index("s")
    wid = c * NS + s

    def _fill_z(i, _):
        zbuf[pl.ds(i * 16, 16)] = jnp.zeros((16,), jnp.float32)
        return 0

    lax.fori_loop(0, RPT // 16, _fill_z, 0)

    def _fill_o(i, _):
        ones_v[pl.ds(i * 16, 16)] = jnp.ones((16,), jnp.float32)
        return 0

    lax.fori_loop(0, CHUNK // 16, _fill_o, 0)

    pltpu.sync_copy(zbuf, acc.at[pl.ds(s * RPT, RPT)])
    plsc.subcore_barrier()

    base = wid * CPT

    def _win(w, _):
        start = pl.multiple_of(base + w * KB, KB)
        pltpu.sync_copy(dst_hbm.at[pl.ds(start, KB)], dst_v)
        for j in range(KB):
            pltpu.sync_copy(ones_v, acc.at[dst_v.at[j]], add=True)
        return 0

    lax.fori_loop(0, SW, _win, 0)

    plsc.subcore_barrier()
    pltpu.sync_copy(acc.at[pl.ds(s * RPT, RPT)],
                    out_hbm.at[c, pl.ds(s * RPT, RPT)])


@functools.partial(
    pl.kernel,
    out_type=jax.ShapeDtypeStruct((NC, NP, D), jnp.float32),
    mesh=_MESH,
    scratch_types=[
        pltpu.VMEM((KB, CHUNK), jnp.int32),
        pltpu.VMEM((KB, CHUNK), jnp.int32),
        pltpu.VMEM((CHUNK, D), jnp.float32),
        pltpu.VMEM((RPT // 5, D), jnp.float32),
        pltpu.VMEM_SHARED((NP, D), jnp.float32),
        pltpu.SemaphoreType.DMA,
    ],
)
def _sc_scatter(y_hbm, src_hbm, dst_hbm, out_hbm,
                src_v, dst_v, rows, zbuf, acc, sem):
    c = lax.axis_index("c")
    s = lax.axis_index("s")
    wid = c * NS + s
    zr = RPT // 5

    def _fill_z(i, _):
        zbuf[i // (D // 16), pl.ds((i % (D // 16)) * 16, 16)] = (
            jnp.zeros((16,), jnp.float32))
        return 0

    lax.fori_loop(0, zr * (D // 16), _fill_z, 0)

    def _zero(j, _):
        pltpu.sync_copy(zbuf, acc.at[pl.ds(s * RPT + j * zr, zr)])
        return 0

    lax.fori_loop(0, RPT // zr, _zero, 0)
    plsc.subcore_barrier()

    base = wid * CPT

    def _win(w, _):
        start = pl.multiple_of(base + w * KB, KB)
        pltpu.sync_copy(src_hbm.at[pl.ds(start, KB)], src_v)
        pltpu.sync_copy(dst_hbm.at[pl.ds(start, KB)], dst_v)
        for j in range(KB):
            pltpu.async_copy(y_hbm.at[src_v.at[j]], rows, sem).wait()
            pltpu.sync_copy(rows, acc.at[dst_v.at[j]], add=True)
        return 0

    lax.fori_loop(0, SW, _win, 0)

    plsc.subcore_barrier()
    pltpu.sync_copy(acc.at[pl.ds(s * RPT, RPT)],
                    out_hbm.at[c, pl.ds(s * RPT, RPT)])


RB = 1000


def _mm_body(x_ref, w_ref, o_ref):
    o_ref[...] = jnp.dot(x_ref[...], w_ref[...],
                         preferred_element_type=jnp.float32)


def _tc_matmul(x, w):
    return pl.pallas_call(
        _mm_body,
        grid=(N // RB,),
        in_specs=[pl.BlockSpec((RB, D), lambda i: (i, 0)),
                  pl.BlockSpec((D, D), lambda i: (0, 0))],
        out_specs=pl.BlockSpec((RB, D), lambda i: (i, 0)),
        out_shape=jax.ShapeDtypeStruct((N, D), jnp.float32),
    )(x, w)


def _scale_body(xw_ref, da_ref, db_ref, y_ref, dinv_ref):
    deg = da_ref[...] + db_ref[...] + 1.0
    dinv = lax.rsqrt(deg)
    y_ref[...] = dinv * xw_ref[...]
    dinv_ref[...] = dinv


def _tc_scale(xw, da, db):
    return pl.pallas_call(
        _scale_body,
        grid=(N // RB,),
        in_specs=[pl.BlockSpec((RB, D), lambda i: (i, 0)),
                  pl.BlockSpec((RB, 1), lambda i: (i, 0)),
                  pl.BlockSpec((RB, 1), lambda i: (i, 0))],
        out_specs=[pl.BlockSpec((RB, D), lambda i: (i, 0)),
                   pl.BlockSpec((RB, 1), lambda i: (i, 0))],
        out_shape=[jax.ShapeDtypeStruct((N, D), jnp.float32),
                   jax.ShapeDtypeStruct((N, 1), jnp.float32)],
    )(xw, da, db)


def _layer_body(sp_ref, y1_ref, dinv_ref, b_ref, w_ref, o_ref):
    sagg = sp_ref[0] + sp_ref[1] + y1_ref[...]
    h = jnp.maximum(dinv_ref[...] * sagg + b_ref[...], 0.0)
    o_ref[...] = dinv_ref[...] * jnp.dot(h, w_ref[...],
                                         preferred_element_type=jnp.float32)


def _tc_layer(sp, y1, dinv, b, w):
    return pl.pallas_call(
        _layer_body,
        grid=(N // RB,),
        in_specs=[pl.BlockSpec((NC, RB, D), lambda i: (0, i, 0)),
                  pl.BlockSpec((RB, D), lambda i: (i, 0)),
                  pl.BlockSpec((RB, 1), lambda i: (i, 0)),
                  pl.BlockSpec((1, D), lambda i: (0, 0)),
                  pl.BlockSpec((D, D), lambda i: (0, 0))],
        out_specs=pl.BlockSpec((RB, D), lambda i: (i, 0)),
        out_shape=jax.ShapeDtypeStruct((N, D), jnp.float32),
    )(sp, y1, dinv, b, w)


def _final_body(sp_ref, y2_ref, dinv_ref, b_ref, o_ref):
    o = dinv_ref[...] * (sp_ref[0] + sp_ref[1] + y2_ref[...]) + b_ref[...]
    m = jnp.max(o, axis=1, keepdims=True)
    z = o - m
    o_ref[...] = z - jnp.log(jnp.sum(jnp.exp(z), axis=1, keepdims=True))


def _tc_final(sp, y2, dinv, b):
    return pl.pallas_call(
        _final_body,
        grid=(N // RB,),
        in_specs=[pl.BlockSpec((NC, RB, D), lambda i: (0, i, 0)),
                  pl.BlockSpec((RB, D), lambda i: (i, 0)),
                  pl.BlockSpec((RB, 1), lambda i: (i, 0)),
                  pl.BlockSpec((1, D), lambda i: (0, 0))],
        out_specs=pl.BlockSpec((RB, D), lambda i: (i, 0)),
        out_shape=jax.ShapeDtypeStruct((N, D), jnp.float32),
    )(sp, y2, dinv, b)


def kernel(x, edge_index, W1, b1, W2, b2):
    src = edge_index[0]
    dst = edge_index[1]
    npad = EP - E
    pi = jnp.arange(npad, dtype=jnp.int32)
    src_p = jnp.concatenate([src, pi % N]).reshape(EP // CHUNK, CHUNK)
    dst_p = jnp.concatenate([dst, N + pi % (NP - N)]).reshape(EP // CHUNK, CHUNK)

    degp = _sc_deg(dst_p)
    xw1 = _tc_matmul(x, W1)
    da = degp[0, :N, None]
    db = degp[1, :N, None]
    y1, dinv = _tc_scale(xw1, da, db)

    sp1 = _sc_scatter(y1, src_p, dst_p)
    y2 = _tc_layer(sp1, y1, dinv, b1.reshape(1, D), W2)

    sp2 = _sc_scatter(y2, src_p, dst_p)
    return _tc_final(sp2, y2, dinv, b2.reshape(1, D))


# speedup vs baseline: 30.2036x; 1.3827x over previous
import functools

import jax
import jax.numpy as jnp
from jax import lax
from jax.experimental import pallas as pl
from jax.experimental.pallas import tpu as pltpu
from jax.experimental.pallas import tpu_sc as plsc

N = 10000
E = 320000
D = 128
NC = 2
NS = 16
NW = NC * NS
NP = 10240
RPT = NP // NS

CHUNK = 128
KB = 16
EP = 327680
CPT = EP // (NW * CHUNK)
SW = CPT // KB

_MESH = plsc.VectorSubcoreMesh(
    core_axis_name="c", subcore_axis_name="s", num_cores=NC, num_subcores=NS
)


@functools.partial(
    pl.kernel,
    out_type=jax.ShapeDtypeStruct((NC, NP), jnp.float32),
    mesh=_MESH,
    scratch_types=[
        pltpu.VMEM((KB, CHUNK), jnp.int32),
        pltpu.VMEM((CHUNK,), jnp.float32),
        pltpu.VMEM((RPT,), jnp.float32),
        pltpu.VMEM_SHARED((NP,), jnp.float32)
    ],
)
def _sc_deg(dst_hbm, out_hbm, dst_v, ones_v, zbuf, acc):
    c = lax.axis_index("c")
    s = lax.axis_index("s")
    wid = c * NS + s

    def _fill_z(i, _):
        zbuf[pl.ds(i * 16, 16)] = jnp.zeros((16,), jnp.float32)
        return 0

    lax.fori_loop(0, RPT // 16, _fill_z, 0)

    def _fill_o(i, _):
        ones_v[pl.ds(i * 16, 16)] = jnp.ones((16,), jnp.float32)
        return 0

    lax.fori_loop(0, CHUNK // 16, _fill_o, 0)

    pltpu.sync_copy(zbuf, acc.at[pl.ds(s * RPT, RPT)])
    plsc.subcore_barrier()

    base = wid * CPT

    def _win(w, _):
        start = pl.multiple_of(base + w * KB, KB)
        pltpu.sync_copy(dst_hbm.at[pl.ds(start, KB)], dst_v)
        for j in range(KB):
            pltpu.sync_copy(ones_v, acc.at[dst_v.at[j]], add=True)
        return 0

    lax.fori_loop(0, SW, _win, 0)

    plsc.subcore_barrier()
    pltpu.sync_copy(acc.at[pl.ds(s * RPT, RPT)],
                    out_hbm.at[c, pl.ds(s * RPT, RPT)])


@functools.partial(
    pl.kernel,
    out_type=jax.ShapeDtypeStruct((NC, NP, D), jnp.float32),
    mesh=_MESH,
    scratch_types=[
        pltpu.VMEM((KB, CHUNK), jnp.int32),
        pltpu.VMEM((KB, CHUNK), jnp.int32),
        pltpu.VMEM((CHUNK, D), jnp.float32),
        pltpu.VMEM((CHUNK, D), jnp.float32),
        pltpu.VMEM_SHARED((NP, D), jnp.float32),
        pltpu.SemaphoreType.DMA,
        pltpu.SemaphoreType.DMA,
    ],
)
def _sc_scatter(y_hbm, src_hbm, dst_hbm, out_hbm,
                src_v, dst_v, rows0, rows1, acc, sem0, sem1):
    c = lax.axis_index("c")
    s = lax.axis_index("s")
    wid = c * NS + s
    zr = CHUNK

    def _fill_z(i, _):
        rows0[i // (D // 16), pl.ds((i % (D // 16)) * 16, 16)] = (
            jnp.zeros((16,), jnp.float32))
        return 0

    lax.fori_loop(0, zr * (D // 16), _fill_z, 0)

    def _zero(j, _):
        pltpu.sync_copy(rows0, acc.at[pl.ds(s * RPT + j * zr, zr)])
        return 0

    lax.fori_loop(0, RPT // zr, _zero, 0)
    plsc.subcore_barrier()

    base = wid * CPT

    bufs = (rows0, rows1)
    sems = (sem0, sem1)

    def _win(w, _):
        start = pl.multiple_of(base + w * KB, KB)
        pltpu.sync_copy(src_hbm.at[pl.ds(start, KB)], src_v)
        pltpu.sync_copy(dst_hbm.at[pl.ds(start, KB)], dst_v)
        descs = [None, None]
        descs[0] = pltpu.async_copy(y_hbm.at[src_v.at[0]], bufs[0], sems[0])
        for j in range(KB):
            if j + 1 < KB:
                descs[(j + 1) % 2] = pltpu.async_copy(
                    y_hbm.at[src_v.at[j + 1]], bufs[(j + 1) % 2],
                    sems[(j + 1) % 2])
            descs[j % 2].wait()
            pltpu.sync_copy(bufs[j % 2], acc.at[dst_v.at[j]], add=True)
        return 0

    lax.fori_loop(0, SW, _win, 0)

    plsc.subcore_barrier()
    pltpu.sync_copy(acc.at[pl.ds(s * RPT, RPT)],
                    out_hbm.at[c, pl.ds(s * RPT, RPT)])


RB = 1000


def _mm_body(x_ref, w_ref, o_ref):
    o_ref[...] = jnp.dot(x_ref[...], w_ref[...],
                         preferred_element_type=jnp.float32)


def _tc_matmul(x, w):
    return pl.pallas_call(
        _mm_body,
        grid=(N // RB,),
        in_specs=[pl.BlockSpec((RB, D), lambda i: (i, 0)),
                  pl.BlockSpec((D, D), lambda i: (0, 0))],
        out_specs=pl.BlockSpec((RB, D), lambda i: (i, 0)),
        out_shape=jax.ShapeDtypeStruct((N, D), jnp.float32),
    )(x, w)


def _scale_body(xw_ref, da_ref, db_ref, y_ref, dinv_ref):
    deg = da_ref[...] + db_ref[...] + 1.0
    dinv = lax.rsqrt(deg)
    y_ref[...] = dinv * xw_ref[...]
    dinv_ref[...] = dinv


def _tc_scale(xw, da, db):
    return pl.pallas_call(
        _scale_body,
        grid=(N // RB,),
        in_specs=[pl.BlockSpec((RB, D), lambda i: (i, 0)),
                  pl.BlockSpec((RB, 1), lambda i: (i, 0)),
                  pl.BlockSpec((RB, 1), lambda i: (i, 0))],
        out_specs=[pl.BlockSpec((RB, D), lambda i: (i, 0)),
                   pl.BlockSpec((RB, 1), lambda i: (i, 0))],
        out_shape=[jax.ShapeDtypeStruct((N, D), jnp.float32),
                   jax.ShapeDtypeStruct((N, 1), jnp.float32)],
    )(xw, da, db)


def _layer_body(sp_ref, y1_ref, dinv_ref, b_ref, w_ref, o_ref):
    sagg = sp_ref[0] + sp_ref[1] + y1_ref[...]
    h = jnp.maximum(dinv_ref[...] * sagg + b_ref[...], 0.0)
    o_ref[...] = dinv_ref[...] * jnp.dot(h, w_ref[...],
                                         preferred_element_type=jnp.float32)


def _tc_layer(sp, y1, dinv, b, w):
    return pl.pallas_call(
        _layer_body,
        grid=(N // RB,),
        in_specs=[pl.BlockSpec((NC, RB, D), lambda i: (0, i, 0)),
                  pl.BlockSpec((RB, D), lambda i: (i, 0)),
                  pl.BlockSpec((RB, 1), lambda i: (i, 0)),
                  pl.BlockSpec((1, D), lambda i: (0, 0)),
                  pl.BlockSpec((D, D), lambda i: (0, 0))],
        out_specs=pl.BlockSpec((RB, D), lambda i: (i, 0)),
        out_shape=jax.ShapeDtypeStruct((N, D), jnp.float32),
    )(sp, y1, dinv, b, w)


def _final_body(sp_ref, y2_ref, dinv_ref, b_ref, o_ref):
    o = dinv_ref[...] * (sp_ref[0] + sp_ref[1] + y2_ref[...]) + b_ref[...]
    m = jnp.max(o, axis=1, keepdims=True)
    z = o - m
    o_ref[...] = z - jnp.log(jnp.sum(jnp.exp(z), axis=1, keepdims=True))


def _tc_final(sp, y2, dinv, b):
    return pl.pallas_call(
        _final_body,
        grid=(N // RB,),
        in_specs=[pl.BlockSpec((NC, RB, D), lambda i: (0, i, 0)),
                  pl.BlockSpec((RB, D), lambda i: (i, 0)),
                  pl.BlockSpec((RB, 1), lambda i: (i, 0)),
                  pl.BlockSpec((1, D), lambda i: (0, 0))],
        out_specs=pl.BlockSpec((RB, D), lambda i: (i, 0)),
        out_shape=jax.ShapeDtypeStruct((N, D), jnp.float32),
    )(sp, y2, dinv, b)


def kernel(x, edge_index, W1, b1, W2, b2):
    src = edge_index[0]
    dst = edge_index[1]
    npad = EP - E
    pi = jnp.arange(npad, dtype=jnp.int32)
    src_p = jnp.concatenate([src, pi % N]).reshape(EP // CHUNK, CHUNK)
    dst_p = jnp.concatenate([dst, N + pi % (NP - N)]).reshape(EP // CHUNK, CHUNK)

    degp = _sc_deg(dst_p)
    xw1 = _tc_matmul(x, W1)
    da = degp[0, :N, None]
    db = degp[1, :N, None]
    y1, dinv = _tc_scale(xw1, da, db)

    sp1 = _sc_scatter(y1, src_p, dst_p)
    y2 = _tc_layer(sp1, y1, dinv, b1.reshape(1, D), W2)

    sp2 = _sc_scatter(y2, src_p, dst_p)
    return _tc_final(sp2, y2, dinv, b2.reshape(1, D))


# speedup vs baseline: 30.3069x; 1.0034x over previous
import functools

import jax
import jax.numpy as jnp
from jax import lax
from jax.experimental import pallas as pl
from jax.experimental.pallas import tpu as pltpu
from jax.experimental.pallas import tpu_sc as plsc

N = 10000
E = 320000
D = 128
NC = 2
NS = 16
NW = NC * NS
NP = 10240
RPT = NP // NS

CHUNK = 128
KB = 16
EP = 327680
CPT = EP // (NW * CHUNK)
SW = CPT // KB

_MESH = plsc.VectorSubcoreMesh(
    core_axis_name="c", subcore_axis_name="s", num_cores=NC, num_subcores=NS
)


@functools.partial(
    pl.kernel,
    out_type=jax.ShapeDtypeStruct((NC, NP), jnp.float32),
    mesh=_MESH,
    scratch_types=[
        pltpu.VMEM((KB, CHUNK), jnp.int32),
        pltpu.VMEM((CHUNK,), jnp.float32),
        pltpu.VMEM((RPT,), jnp.float32),
        pltpu.VMEM_SHARED((NP,), jnp.float32)
    ],
)
def _sc_deg(dst_hbm, out_hbm, dst_v, ones_v, zbuf, acc):
    c = lax.axis_index("c")
    s = lax.axis_index("s")
    wid = c * NS + s

    def _fill_z(i, _):
        zbuf[pl.ds(i * 16, 16)] = jnp.zeros((16,), jnp.float32)
        return 0

    lax.fori_loop(0, RPT // 16, _fill_z, 0)

    def _fill_o(i, _):
        ones_v[pl.ds(i * 16, 16)] = jnp.ones((16,), jnp.float32)
        return 0

    lax.fori_loop(0, CHUNK // 16, _fill_o, 0)

    pltpu.sync_copy(zbuf, acc.at[pl.ds(s * RPT, RPT)])
    plsc.subcore_barrier()

    base = wid * CPT

    def _win(w, _):
        start = pl.multiple_of(base + w * KB, KB)
        pltpu.sync_copy(dst_hbm.at[pl.ds(start, KB)], dst_v)
        for j in range(KB):
            pltpu.sync_copy(ones_v, acc.at[dst_v.at[j]], add=True)
        return 0

    lax.fori_loop(0, SW, _win, 0)

    plsc.subcore_barrier()
    pltpu.sync_copy(acc.at[pl.ds(s * RPT, RPT)],
                    out_hbm.at[c, pl.ds(s * RPT, RPT)])


@functools.partial(
    pl.kernel,
    out_type=jax.ShapeDtypeStruct((NC, NP, D), jnp.float32),
    mesh=_MESH,
    scratch_types=[
        pltpu.VMEM((KB, CHUNK), jnp.int32),
        pltpu.VMEM((KB, CHUNK), jnp.int32),
        pltpu.VMEM((CHUNK, D), jnp.float32),
        pltpu.VMEM((CHUNK, D), jnp.float32),
        pltpu.VMEM_SHARED((NP, D), jnp.float32),
        pltpu.SemaphoreType.DMA,
        pltpu.SemaphoreType.DMA,
    ],
)
def _sc_scatter(y_hbm, src_hbm, dst_hbm, out_hbm,
                src_v, dst_v, rows0, rows1, acc, sem0, sem1):
    c = lax.axis_index("c")
    s = lax.axis_index("s")
    wid = c * NS + s
    zr = CHUNK
    base = wid * CPT

    bufs = (rows1, rows0)
    sems = (sem1, sem0)

    pltpu.sync_copy(src_hbm.at[pl.ds(pl.multiple_of(base, KB), KB)], src_v)
    pltpu.sync_copy(dst_hbm.at[pl.ds(pl.multiple_of(base, KB), KB)], dst_v)
    pltpu.async_copy(y_hbm.at[src_v.at[0]], rows1, sem1)

    def _fill_z(i, _):
        rows0[i // (D // 16), pl.ds((i % (D // 16)) * 16, 16)] = (
            jnp.zeros((16,), jnp.float32))
        return 0

    lax.fori_loop(0, zr * (D // 16), _fill_z, 0)

    def _zero(j, _):
        pltpu.sync_copy(rows0, acc.at[pl.ds(s * RPT + j * zr, zr)])
        return 0

    lax.fori_loop(0, RPT // zr, _zero, 0)
    plsc.subcore_barrier()

    def _chunks(skip_first_load):
        descs = [None, None]
        if not skip_first_load:
            descs[0] = pltpu.async_copy(y_hbm.at[src_v.at[0]], bufs[0],
                                        sems[0])
        for j in range(KB):
            if j + 1 < KB:
                descs[(j + 1) % 2] = pltpu.async_copy(
                    y_hbm.at[src_v.at[j + 1]], bufs[(j + 1) % 2],
                    sems[(j + 1) % 2])
            if j == 0 and skip_first_load:
                pltpu.make_async_copy(y_hbm.at[src_v.at[0]], bufs[0],
                                      sems[0]).wait()
            else:
                descs[j % 2].wait()
            pltpu.sync_copy(bufs[j % 2], acc.at[dst_v.at[j]], add=True)

    _chunks(True)

    def _win(w, _):
        start = pl.multiple_of(base + w * KB, KB)
        pltpu.sync_copy(src_hbm.at[pl.ds(start, KB)], src_v)
        pltpu.sync_copy(dst_hbm.at[pl.ds(start, KB)], dst_v)
        _chunks(False)
        return 0

    lax.fori_loop(1, SW, _win, 0)

    plsc.subcore_barrier()
    pltpu.sync_copy(acc.at[pl.ds(s * RPT, RPT)],
                    out_hbm.at[c, pl.ds(s * RPT, RPT)])


RB = 1000


def _prescale_body(x_ref, w_ref, da_ref, db_ref, y_ref, dinv_ref):
    deg = da_ref[...] + db_ref[...] + 1.0
    dinv = lax.rsqrt(deg)
    y_ref[...] = dinv * jnp.dot(x_ref[...], w_ref[...],
                                preferred_element_type=jnp.float32)
    dinv_ref[...] = dinv


def _tc_prescale(x, w, da, db):
    return pl.pallas_call(
        _prescale_body,
        grid=(N // RB,),
        in_specs=[pl.BlockSpec((RB, D), lambda i: (i, 0)),
                  pl.BlockSpec((D, D), lambda i: (0, 0)),
                  pl.BlockSpec((RB, 1), lambda i: (i, 0)),
                  pl.BlockSpec((RB, 1), lambda i: (i, 0))],
        out_specs=[pl.BlockSpec((RB, D), lambda i: (i, 0)),
                   pl.BlockSpec((RB, 1), lambda i: (i, 0))],
        out_shape=[jax.ShapeDtypeStruct((N, D), jnp.float32),
                   jax.ShapeDtypeStruct((N, 1), jnp.float32)],
    )(x, w, da, db)


def _layer_body(sp_ref, y1_ref, dinv_ref, b_ref, w_ref, o_ref):
    sagg = sp_ref[0] + sp_ref[1] + y1_ref[...]
    h = jnp.maximum(dinv_ref[...] * sagg + b_ref[...], 0.0)
    o_ref[...] = dinv_ref[...] * jnp.dot(h, w_ref[...],
                                         preferred_element_type=jnp.float32)


def _tc_layer(sp, y1, dinv, b, w):
    return pl.pallas_call(
        _layer_body,
        grid=(N // RB,),
        in_specs=[pl.BlockSpec((NC, RB, D), lambda i: (0, i, 0)),
                  pl.BlockSpec((RB, D), lambda i: (i, 0)),
                  pl.BlockSpec((RB, 1), lambda i: (i, 0)),
                  pl.BlockSpec((1, D), lambda i: (0, 0)),
                  pl.BlockSpec((D, D), lambda i: (0, 0))],
        out_specs=pl.BlockSpec((RB, D), lambda i: (i, 0)),
        out_shape=jax.ShapeDtypeStruct((N, D), jnp.float32),
    )(sp, y1, dinv, b, w)


def _final_body(sp_ref, y2_ref, dinv_ref, b_ref, o_ref):
    o = dinv_ref[...] * (sp_ref[0] + sp_ref[1] + y2_ref[...]) + b_ref[...]
    m = jnp.max(o, axis=1, keepdims=True)
    z = o - m
    o_ref[...] = z - jnp.log(jnp.sum(jnp.exp(z), axis=1, keepdims=True))


def _tc_final(sp, y2, dinv, b):
    return pl.pallas_call(
        _final_body,
        grid=(N // RB,),
        in_specs=[pl.BlockSpec((NC, RB, D), lambda i: (0, i, 0)),
                  pl.BlockSpec((RB, D), lambda i: (i, 0)),
                  pl.BlockSpec((RB, 1), lambda i: (i, 0)),
                  pl.BlockSpec((1, D), lambda i: (0, 0))],
        out_specs=pl.BlockSpec((RB, D), lambda i: (i, 0)),
        out_shape=jax.ShapeDtypeStruct((N, D), jnp.float32),
    )(sp, y2, dinv, b)


def kernel(x, edge_index, W1, b1, W2, b2):
    src = edge_index[0]
    dst = edge_index[1]
    npad = EP - E
    pi = jnp.arange(npad, dtype=jnp.int32)
    src_p = jnp.concatenate([src, pi % N]).reshape(EP // CHUNK, CHUNK)
    dst_p = jnp.concatenate([dst, N + pi % (NP - N)]).reshape(EP // CHUNK, CHUNK)

    degp = _sc_deg(dst_p)
    da = degp[0, :N, None]
    db = degp[1, :N, None]
    y1, dinv = _tc_prescale(x, W1, da, db)

    sp1 = _sc_scatter(y1, src_p, dst_p)
    y2 = _tc_layer(sp1, y1, dinv, b1.reshape(1, D), W2)

    sp2 = _sc_scatter(y2, src_p, dst_p)
    return _tc_final(sp2, y2, dinv, b2.reshape(1, D))


# speedup vs baseline: 30.3439x; 1.0012x over previous
import functools

import numpy as np
import jax
import jax.numpy as jnp
from jax import lax
from jax.experimental import pallas as pl
from jax.experimental.pallas import tpu as pltpu
from jax.experimental.pallas import tpu_sc as plsc

N = 10000
E = 320000
D = 128
NC = 2
NS = 16
NW = NC * NS
NP = 10240
RPT = NP // NS

CHUNK = 128
KB = 16
EP = 327680
CPT = EP // (NW * CHUNK)
SW = CPT // KB

_MESH = plsc.VectorSubcoreMesh(
    core_axis_name="c", subcore_axis_name="s", num_cores=NC, num_subcores=NS
)


@functools.partial(
    pl.kernel,
    out_type=jax.ShapeDtypeStruct((NC, NP), jnp.float32),
    mesh=_MESH,
    scratch_types=[
        pltpu.VMEM((KB, CHUNK), jnp.int32),
        pltpu.VMEM((CHUNK,), jnp.float32),
        pltpu.VMEM((RPT,), jnp.float32),
        pltpu.VMEM_SHARED((NP,), jnp.float32)
    ],
)
def _sc_deg(dst_hbm, out_hbm, dst_v, ones_v, zbuf, acc):
    c = lax.axis_index("c")
    s = lax.axis_index("s")
    wid = c * NS + s

    def _fill_z(i, _):
        zbuf[pl.ds(i * 16, 16)] = jnp.zeros((16,), jnp.float32)
        return 0

    lax.fori_loop(0, RPT // 16, _fill_z, 0)

    def _fill_o(i, _):
        ones_v[pl.ds(i * 16, 16)] = jnp.ones((16,), jnp.float32)
        return 0

    lax.fori_loop(0, CHUNK // 16, _fill_o, 0)

    pltpu.sync_copy(zbuf, acc.at[pl.ds(s * RPT, RPT)])
    plsc.subcore_barrier()

    base = wid * CPT

    def _win(w, _):
        start = pl.multiple_of(base + w * KB, KB)
        pltpu.sync_copy(dst_hbm.at[pl.ds(start, KB)], dst_v)
        for j in range(KB):
            pltpu.sync_copy(ones_v, acc.at[dst_v.at[j]], add=True)
        return 0

    lax.fori_loop(0, SW, _win, 0)

    plsc.subcore_barrier()
    pltpu.sync_copy(acc.at[pl.ds(s * RPT, RPT)],
                    out_hbm.at[c, pl.ds(s * RPT, RPT)])


@functools.partial(
    pl.kernel,
    out_type=jax.ShapeDtypeStruct((NC, NP, D), jnp.float32),
    mesh=_MESH,
    scratch_types=[
        pltpu.VMEM((KB, CHUNK), jnp.int32),
        pltpu.VMEM((KB, CHUNK), jnp.int32),
        pltpu.VMEM((CHUNK, D), jnp.float32),
        pltpu.VMEM((CHUNK, D), jnp.float32),
        pltpu.VMEM_SHARED((NP, D), jnp.float32),
        pltpu.SemaphoreType.DMA,
        pltpu.SemaphoreType.DMA,
    ],
)
def _sc_scatter(y_hbm, src_hbm, dst_hbm, out_hbm,
                src_v, dst_v, rows0, rows1, acc, sem0, sem1):
    c = lax.axis_index("c")
    s = lax.axis_index("s")
    wid = c * NS + s
    zr = CHUNK
    base = wid * CPT

    bufs = (rows1, rows0)
    sems = (sem1, sem0)

    pltpu.sync_copy(src_hbm.at[pl.ds(pl.multiple_of(base, KB), KB)], src_v)
    pltpu.sync_copy(dst_hbm.at[pl.ds(pl.multiple_of(base, KB), KB)], dst_v)
    pltpu.async_copy(y_hbm.at[src_v.at[0]], rows1, sem1)

    def _fill_z(i, _):
        rows0[i // (D // 16), pl.ds((i % (D // 16)) * 16, 16)] = (
            jnp.zeros((16,), jnp.float32))
        return 0

    lax.fori_loop(0, zr * (D // 16), _fill_z, 0)

    def _zero(j, _):
        pltpu.sync_copy(rows0, acc.at[pl.ds(s * RPT + j * zr, zr)])
        return 0

    lax.fori_loop(0, RPT // zr, _zero, 0)
    plsc.subcore_barrier()

    def _chunks(skip_first_load):
        descs = [None, None]
        if not skip_first_load:
            descs[0] = pltpu.async_copy(y_hbm.at[src_v.at[0]], bufs[0],
                                        sems[0])
        for j in range(KB):
            if j + 1 < KB:
                descs[(j + 1) % 2] = pltpu.async_copy(
                    y_hbm.at[src_v.at[j + 1]], bufs[(j + 1) % 2],
                    sems[(j + 1) % 2])
            if j == 0 and skip_first_load:
                pltpu.make_async_copy(y_hbm.at[src_v.at[0]], bufs[0],
                                      sems[0]).wait()
            else:
                descs[j % 2].wait()
            pltpu.sync_copy(bufs[j % 2], acc.at[dst_v.at[j]], add=True)

    _chunks(True)

    def _win(w, _):
        start = pl.multiple_of(base + w * KB, KB)
        pltpu.sync_copy(src_hbm.at[pl.ds(start, KB)], src_v)
        pltpu.sync_copy(dst_hbm.at[pl.ds(start, KB)], dst_v)
        _chunks(False)
        return 0

    lax.fori_loop(1, SW, _win, 0)

    plsc.subcore_barrier()
    pltpu.sync_copy(acc.at[pl.ds(s * RPT, RPT)],
                    out_hbm.at[c, pl.ds(s * RPT, RPT)])


RB = 1000


def _prescale_body(x_ref, w_ref, da_ref, db_ref, y_ref, dinv_ref):
    deg = da_ref[...] + db_ref[...] + 1.0
    dinv = lax.rsqrt(deg)
    y_ref[...] = dinv * jnp.dot(x_ref[...], w_ref[...],
                                preferred_element_type=jnp.float32)
    dinv_ref[...] = dinv


def _tc_prescale(x, w, da, db):
    return pl.pallas_call(
        _prescale_body,
        grid=(N // RB,),
        in_specs=[pl.BlockSpec((RB, D), lambda i: (i, 0)),
                  pl.BlockSpec((D, D), lambda i: (0, 0)),
                  pl.BlockSpec((RB, 1), lambda i: (i, 0)),
                  pl.BlockSpec((RB, 1), lambda i: (i, 0))],
        out_specs=[pl.BlockSpec((RB, D), lambda i: (i, 0)),
                   pl.BlockSpec((RB, 1), lambda i: (i, 0))],
        out_shape=[jax.ShapeDtypeStruct((N, D), jnp.float32),
                   jax.ShapeDtypeStruct((N, 1), jnp.float32)],
    )(x, w, da, db)


def _layer_body(sp_ref, y1_ref, dinv_ref, b_ref, w_ref, o_ref):
    sagg = sp_ref[0] + sp_ref[1] + y1_ref[...]
    h = jnp.maximum(dinv_ref[...] * sagg + b_ref[...], 0.0)
    o_ref[...] = dinv_ref[...] * jnp.dot(h, w_ref[...],
                                         preferred_element_type=jnp.float32)


def _tc_layer(sp, y1, dinv, b, w):
    return pl.pallas_call(
        _layer_body,
        grid=(N // RB,),
        in_specs=[pl.BlockSpec((NC, RB, D), lambda i: (0, i, 0)),
                  pl.BlockSpec((RB, D), lambda i: (i, 0)),
                  pl.BlockSpec((RB, 1), lambda i: (i, 0)),
                  pl.BlockSpec((1, D), lambda i: (0, 0)),
                  pl.BlockSpec((D, D), lambda i: (0, 0))],
        out_specs=pl.BlockSpec((RB, D), lambda i: (i, 0)),
        out_shape=jax.ShapeDtypeStruct((N, D), jnp.float32),
    )(sp, y1, dinv, b, w)


def _final_body(sp_ref, y2_ref, dinv_ref, b_ref, o_ref):
    o = dinv_ref[...] * (sp_ref[0] + sp_ref[1] + y2_ref[...]) + b_ref[...]
    m = jnp.max(o, axis=1, keepdims=True)
    z = o - m
    o_ref[...] = z - jnp.log(jnp.sum(jnp.exp(z), axis=1, keepdims=True))


def _tc_final(sp, y2, dinv, b):
    return pl.pallas_call(
        _final_body,
        grid=(N // RB,),
        in_specs=[pl.BlockSpec((NC, RB, D), lambda i: (0, i, 0)),
                  pl.BlockSpec((RB, D), lambda i: (i, 0)),
                  pl.BlockSpec((RB, 1), lambda i: (i, 0)),
                  pl.BlockSpec((1, D), lambda i: (0, 0))],
        out_specs=pl.BlockSpec((RB, D), lambda i: (i, 0)),
        out_shape=jax.ShapeDtypeStruct((N, D), jnp.float32),
    )(sp, y2, dinv, b)


def kernel(x, edge_index, W1, b1, W2, b2):
    src = edge_index[0]
    dst = edge_index[1]
    npad = EP - E
    pi = np.arange(npad, dtype=np.int32)
    src_pad = jnp.asarray(pi % N)
    dst_pad = jnp.asarray(N + pi % (NP - N))
    src_p = jnp.concatenate([src, src_pad]).reshape(EP // CHUNK, CHUNK)
    dst_p = jnp.concatenate([dst, dst_pad]).reshape(EP // CHUNK, CHUNK)

    degp = _sc_deg(dst_p)
    da = degp[0, :N, None]
    db = degp[1, :N, None]
    y1, dinv = _tc_prescale(x, W1, da, db)

    sp1 = _sc_scatter(y1, src_p, dst_p)
    y2 = _tc_layer(sp1, y1, dinv, b1.reshape(1, D), W2)

    sp2 = _sc_scatter(y2, src_p, dst_p)
    return _tc_final(sp2, y2, dinv, b2.reshape(1, D))
